# Initial kernel scaffold; baseline (speedup 1.0000x reference)
import jax
import jax.numpy as jnp
from jax.experimental import pallas as pl


def kernel(indices, table_0, table_1, table_2, table_3):
    raise NotImplementedError("write your pallas kernel here")


# speedup vs baseline: 1.6175x; 1.6175x over previous
import jax
import jax.numpy as jnp
from jax import lax
from jax.experimental import pallas as pl
from jax.experimental.pallas import tpu as pltpu
from jax.experimental.pallas import tpu_sc as plsc

_BATCH = 16384
_FIELDS = 26
_CHUNK_OUT = 32
_N_CHUNKS = 4
_OUT_DIM = _N_CHUNKS * _CHUNK_OUT
_B_FLAT = _BATCH * _FIELDS
_L = 128
_NW = 32
_ROWS_PER_W = _B_FLAT // (_NW * _L)
_TROWS = 1000000 // _N_CHUNKS

_mesh = plsc.VectorSubcoreMesh(core_axis_name="core", subcore_axis_name="subcore")


@jax.jit
def kernel(indices, table_0, table_1, table_2, table_3):
    idx = indices.reshape(_B_FLAT // _L, _L).astype(jnp.int32)
    tv = [t.reshape(_TROWS, _OUT_DIM) for t in (table_0, table_1, table_2, table_3)]

    @pl.kernel(
        out_type=jax.ShapeDtypeStruct((_B_FLAT, _OUT_DIM), jnp.float32),
        mesh=_mesh,
        scratch_types=[
            pltpu.VMEM((_L,), jnp.int32),
            pltpu.VMEM((_L,), jnp.int32),
            pltpu.VMEM((_L, _OUT_DIM), jnp.float32),
            pltpu.VMEM((_L, _OUT_DIM), jnp.float32),
            pltpu.VMEM((_L, _OUT_DIM), jnp.float32),
            pltpu.VMEM((_L, _OUT_DIM), jnp.float32),
            pltpu.VMEM((_L, _OUT_DIM), jnp.float32),
            pltpu.SemaphoreType.DMA,
        ],
    )
    def k(idx_hbm, t0_hbm, t1_hbm, t2_hbm, t3_hbm, o_hbm,
          idx_v, q_v, gb0, gb1, gb2, gb3, obuf, sem):
        tables = (t0_hbm, t1_hbm, t2_hbm, t3_hbm)
        gbs = (gb0, gb1, gb2, gb3)
        wid = lax.axis_index("subcore") * 2 + lax.axis_index("core")
        row0 = wid * _ROWS_PER_W

        @pl.loop(0, _ROWS_PER_W)
        def _(c):
            irow = row0 + c
            pltpu.sync_copy(idx_hbm.at[irow], idx_v)
            for v in range(_L // 16):
                q_v[pl.ds(v * 16, 16)] = idx_v[pl.ds(v * 16, 16)] >> 2
            copies = [
                pltpu.async_copy(tables[t].at[q_v], gbs[t], sem)
                for t in range(_N_CHUNKS)
            ]
            for cp in copies:
                cp.wait()

            @pl.loop(0, _L // 16)
            def _(g):
                offs = (idx_v[pl.ds(g * 16, 16)] & 3) * _CHUNK_OUT
                for l in range(16):
                    r = g * 16 + l
                    off = offs[l]
                    for t in range(_N_CHUNKS):
                        for h in range(_CHUNK_OUT // 16):
                            obuf[r, pl.ds(t * _CHUNK_OUT + h * 16, 16)] = gbs[
                                t
                            ][r, pl.ds(off + h * 16, 16)]

            pltpu.sync_copy(obuf, o_hbm.at[pl.ds(irow * _L, _L)])

    out = k(idx, *tv)
    return out.reshape(_BATCH, _FIELDS, _OUT_DIM)


# speedup vs baseline: 1.8151x; 1.1222x over previous
import jax
import jax.numpy as jnp
from jax import lax
from jax.experimental import pallas as pl
from jax.experimental.pallas import tpu as pltpu
from jax.experimental.pallas import tpu_sc as plsc

_BATCH = 16384
_FIELDS = 26
_CHUNK_OUT = 32
_N_CHUNKS = 4
_OUT_DIM = _N_CHUNKS * _CHUNK_OUT
_B_FLAT = _BATCH * _FIELDS
_L = 128
_NW = 32
_ROWS_PER_W = _B_FLAT // (_NW * _L)
_TROWS = 1000000 // _N_CHUNKS

_mesh = plsc.VectorSubcoreMesh(core_axis_name="core", subcore_axis_name="subcore")


@jax.jit
def kernel(indices, table_0, table_1, table_2, table_3):
    idx = indices.reshape(_B_FLAT // _L, _L).astype(jnp.int32)
    tv = [t.reshape(_TROWS, _OUT_DIM) for t in (table_0, table_1, table_2, table_3)]

    @pl.kernel(
        out_type=jax.ShapeDtypeStruct((_B_FLAT, _OUT_DIM), jnp.float32),
        mesh=_mesh,
        scratch_types=[
            pltpu.VMEM((2, _L), jnp.int32),
            pltpu.VMEM((2, _L), jnp.int32),
            pltpu.VMEM((_L, _OUT_DIM), jnp.float32),
            pltpu.VMEM((_L, _OUT_DIM), jnp.float32),
            pltpu.VMEM((2, _L, _OUT_DIM), jnp.float32),
            pltpu.SemaphoreType.DMA,
            pltpu.SemaphoreType.DMA,
            pltpu.SemaphoreType.DMA,
            pltpu.SemaphoreType.DMA,
            pltpu.SemaphoreType.DMA,
        ],
    )
    def k(idx_hbm, t0_hbm, t1_hbm, t2_hbm, t3_hbm, o_hbm,
          idx_v, q_v, gb0, gb1, obuf, sg0, sg1, si, so0, so1):
        tables = (t0_hbm, t1_hbm, t2_hbm, t3_hbm)
        gbs = (gb0, gb1)
        sgs = (sg0, sg1)
        sos = (so0, so1)
        wid = lax.axis_index("subcore") * 2 + lax.axis_index("core")
        row0 = wid * _ROWS_PER_W

        def compute_q(p):
            for v in range(_L // 16):
                q_v[p, pl.ds(v * 16, 16)] = idx_v[p, pl.ds(v * 16, 16)] >> 2

        def assemble(gb, p, t):
            @pl.loop(0, _L // 16)
            def _(g):
                offs = (idx_v[p, pl.ds(g * 16, 16)] & 3) * _CHUNK_OUT
                for l in range(16):
                    off = offs[l]
                    for h in range(_CHUNK_OUT // 16):
                        obuf[p, g * 16 + l, pl.ds(t * _CHUNK_OUT + h * 16, 16)] = (
                            gb[g * 16 + l, pl.ds(off + h * 16, 16)]
                        )

        def chunk_body(c, p):
            pn = 1 - p
            irow = row0 + c

            @pl.when(c < _ROWS_PER_W - 1)
            def _():
                pltpu.async_copy(idx_hbm.at[irow + 1], idx_v.at[pn], si)

            @pl.when(c >= 2)
            def _():
                pltpu.make_async_copy(
                    obuf.at[p], o_hbm.at[pl.ds((irow - 2) * _L, _L)], sos[p]
                ).wait()

            pltpu.async_copy(tables[1].at[q_v.at[p]], gb1, sg1)
            pltpu.make_async_copy(tables[0].at[q_v.at[p]], gb0, sg0).wait()
            assemble(gb0, p, 0)

            pltpu.async_copy(tables[2].at[q_v.at[p]], gb0, sg0)
            pltpu.make_async_copy(tables[1].at[q_v.at[p]], gb1, sg1).wait()
            assemble(gb1, p, 1)

            pltpu.async_copy(tables[3].at[q_v.at[p]], gb1, sg1)
            pltpu.make_async_copy(tables[2].at[q_v.at[p]], gb0, sg0).wait()
            assemble(gb0, p, 2)

            @pl.when(c < _ROWS_PER_W - 1)
            def _():
                pltpu.make_async_copy(idx_hbm.at[irow + 1], idx_v.at[pn], si).wait()
                compute_q(pn)
                pltpu.async_copy(tables[0].at[q_v.at[pn]], gb0, sg0)

            pltpu.make_async_copy(tables[3].at[q_v.at[p]], gb1, sg1).wait()
            assemble(gb1, p, 3)

            pltpu.async_copy(obuf.at[p], o_hbm.at[pl.ds(irow * _L, _L)], sos[p])

        pltpu.sync_copy(idx_hbm.at[row0], idx_v.at[0])
        compute_q(0)
        pltpu.async_copy(tables[0].at[q_v.at[0]], gb0, sg0)

        @pl.loop(0, _ROWS_PER_W // 2)
        def _(cc):
            chunk_body(cc * 2, 0)
            chunk_body(cc * 2 + 1, 1)

        last = row0 + _ROWS_PER_W - 1
        pltpu.make_async_copy(
            obuf.at[0], o_hbm.at[pl.ds((last - 1) * _L, _L)], so0
        ).wait()
        pltpu.make_async_copy(
            obuf.at[1], o_hbm.at[pl.ds(last * _L, _L)], so1
        ).wait()

    out = k(idx, *tv)
    return out.reshape(_BATCH, _FIELDS, _OUT_DIM)


# speedup vs baseline: 1.8303x; 1.0084x over previous
import jax
import jax.numpy as jnp
from jax import lax
from jax.experimental import pallas as pl
from jax.experimental.pallas import tpu as pltpu
from jax.experimental.pallas import tpu_sc as plsc

_BATCH = 16384
_FIELDS = 26
_CHUNK_OUT = 32
_N_CHUNKS = 4
_OUT_DIM = _N_CHUNKS * _CHUNK_OUT
_B_FLAT = _BATCH * _FIELDS
_L = 128
_NW = 32
_ROWS_PER_W = _B_FLAT // (_NW * _L)

_mesh = plsc.VectorSubcoreMesh(core_axis_name="core", subcore_axis_name="subcore")


@jax.jit
def kernel(indices, table_0, table_1, table_2, table_3):
    idx = indices.reshape(_B_FLAT // _L, _L).astype(jnp.int32)
    tv = [
        jnp.pad(t, ((0, 0), (0, _OUT_DIM - _CHUNK_OUT)))
        for t in (table_0, table_1, table_2, table_3)
    ]

    @pl.kernel(
        out_type=jax.ShapeDtypeStruct((_B_FLAT, _OUT_DIM), jnp.float32),
        mesh=_mesh,
        scratch_types=[
            pltpu.VMEM((2, _L), jnp.int32),
            pltpu.VMEM((_L, _OUT_DIM), jnp.float32),
            pltpu.VMEM((_L, _OUT_DIM), jnp.float32),
            pltpu.VMEM((2, _L, _OUT_DIM), jnp.float32),
            pltpu.SemaphoreType.DMA,
            pltpu.SemaphoreType.DMA,
            pltpu.SemaphoreType.DMA,
            pltpu.SemaphoreType.DMA,
            pltpu.SemaphoreType.DMA,
        ],
    )
    def k(idx_hbm, t0_hbm, t1_hbm, t2_hbm, t3_hbm, o_hbm,
          idx_v, gb0, gb1, obuf, sg0, sg1, si, so0, so1):
        tables = (t0_hbm, t1_hbm, t2_hbm, t3_hbm)
        sos = (so0, so1)
        wid = lax.axis_index("subcore") * 2 + lax.axis_index("core")
        row0 = wid * _ROWS_PER_W

        def assemble(gb, p, t):
            @pl.loop(0, _L)
            def _(r):
                for h in range(_CHUNK_OUT // 16):
                    obuf[p, r, pl.ds(t * _CHUNK_OUT + h * 16, 16)] = gb[
                        r, pl.ds(h * 16, 16)
                    ]

        def chunk_body(c, p):
            pn = 1 - p
            irow = row0 + c

            @pl.when(c < _ROWS_PER_W - 1)
            def _():
                pltpu.async_copy(idx_hbm.at[irow + 1], idx_v.at[pn], si)

            @pl.when(c >= 2)
            def _():
                pltpu.make_async_copy(
                    obuf.at[p], o_hbm.at[pl.ds((irow - 2) * _L, _L)], sos[p]
                ).wait()

            pltpu.async_copy(tables[1].at[idx_v.at[p]], gb1, sg1)
            pltpu.make_async_copy(tables[0].at[idx_v.at[p]], gb0, sg0).wait()
            assemble(gb0, p, 0)

            pltpu.async_copy(tables[2].at[idx_v.at[p]], gb0, sg0)
            pltpu.make_async_copy(tables[1].at[idx_v.at[p]], gb1, sg1).wait()
            assemble(gb1, p, 1)

            pltpu.async_copy(tables[3].at[idx_v.at[p]], gb1, sg1)
            pltpu.make_async_copy(tables[2].at[idx_v.at[p]], gb0, sg0).wait()
            assemble(gb0, p, 2)

            @pl.when(c < _ROWS_PER_W - 1)
            def _():
                pltpu.make_async_copy(idx_hbm.at[irow + 1], idx_v.at[pn], si).wait()
                pltpu.async_copy(tables[0].at[idx_v.at[pn]], gb0, sg0)

            pltpu.make_async_copy(tables[3].at[idx_v.at[p]], gb1, sg1).wait()
            assemble(gb1, p, 3)

            pltpu.async_copy(obuf.at[p], o_hbm.at[pl.ds(irow * _L, _L)], sos[p])

        pltpu.sync_copy(idx_hbm.at[row0], idx_v.at[0])
        pltpu.async_copy(tables[0].at[idx_v.at[0]], gb0, sg0)

        @pl.loop(0, _ROWS_PER_W // 2)
        def _(cc):
            chunk_body(cc * 2, 0)
            chunk_body(cc * 2 + 1, 1)

        last = row0 + _ROWS_PER_W - 1
        pltpu.make_async_copy(
            obuf.at[0], o_hbm.at[pl.ds((last - 1) * _L, _L)], so0
        ).wait()
        pltpu.make_async_copy(
            obuf.at[1], o_hbm.at[pl.ds(last * _L, _L)], so1
        ).wait()

    out = k(idx, *tv)
    return out.reshape(_BATCH, _FIELDS, _OUT_DIM)
